# baseline (speedup 1.0000x reference)
import jax
import jax.numpy as jnp
from jax.experimental import pallas as pl

B = 1024
N_PATCHES = 128
D_MODEL = 768
BB = 32


def _body(rp_ref, w_ref, s_ref, out_ref):
    wv = (w_ref[...] * s_ref[0, 0]).reshape(1, 1, D_MODEL)
    out_ref[...] = rp_ref[...][:, :, None] * wv


def kernel(n_patches, relative_positions, W, scale):
    w2 = W.reshape(1, D_MODEL)
    s2 = scale.reshape(1, 1)
    grid = (B // BB,)
    out = pl.pallas_call(
        _body,
        grid=grid,
        in_specs=[
            pl.BlockSpec((BB, N_PATCHES), lambda i: (i, 0)),
            pl.BlockSpec((1, D_MODEL), lambda i: (0, 0)),
            pl.BlockSpec((1, 1), lambda i: (0, 0)),
        ],
        out_specs=pl.BlockSpec((BB, N_PATCHES, D_MODEL), lambda i: (i, 0, 0)),
        out_shape=jax.ShapeDtypeStruct((B, N_PATCHES, D_MODEL), jnp.float32),
    )(relative_positions, w2, s2)
    return out
